# Initial kernel scaffold; baseline (speedup 1.0000x reference)
import jax
import jax.numpy as jnp
from jax.experimental import pallas as pl


def kernel(inputs, edge_index, W0, al0, ar0, W1, al1, ar1, W2, al2, ar2):
    raise NotImplementedError("write your pallas kernel here")


# speedup vs baseline: 55.5599x; 55.5599x over previous
import functools

import jax
import jax.numpy as jnp
from jax import lax
from jax.experimental import pallas as pl
from jax.experimental.pallas import tpu as pltpu
from jax.experimental.pallas import tpu_sc as plsc

N = 10000
NP = 10240
E = 320000
NC = 2
NS = 16
L = 16
NW = NC * NS
EPW = E // NW
B = 80
NB = EPW // B
RPT = NP // NS
ZR = 40


def _sc_scratch(FW, PW):
    CW = FW + 16
    return [
        pltpu.VMEM((B,), jnp.int32),
        pltpu.VMEM((B,), jnp.int32),
        pltpu.VMEM((B, CW), jnp.float32),
        pltpu.VMEM((B, L), jnp.float32),
        pltpu.VMEM((B, PW), jnp.float32),
        pltpu.VMEM((ZR, PW), jnp.float32),
        pltpu.VMEM_SHARED((NP, PW), jnp.float32),
        pltpu.SemaphoreType.DMA,
        pltpu.SemaphoreType.DMA,
    ]


def _sc_body(FW, PW, chunk_heads):
    NCH = len(chunk_heads)

    def body(featx, ert, srcl, dstl, part,
             sidx, didx, fx, erd, pay, zbuf, acc, sem1, sem2):
        c = lax.axis_index("c")
        s = lax.axis_index("s")
        wid = c * NS + s
        zero16 = jnp.zeros((L,), jnp.float32)

        @pl.loop(0, ZR)
        def _(r):
            for k in range(PW // L):
                zbuf[r, pl.ds(k * L, L)] = zero16

        r0 = s * RPT

        @pl.loop(0, RPT // ZR)
        def _(i):
            pltpu.sync_copy(zbuf, acc.at[pl.ds(r0 + i * ZR, ZR), :])

        plsc.subcore_barrier()

        @pl.loop(0, NB)
        def _(j):
            base = wid * EPW + j * B
            pltpu.sync_copy(srcl.at[pl.ds(base, B)], sidx)
            pltpu.sync_copy(dstl.at[pl.ds(base, B)], didx)
            g1 = pltpu.async_copy(featx.at[sidx], fx, sem1)
            g2 = pltpu.async_copy(ert.at[didx], erd, sem2)
            g1.wait()
            g2.wait()

            @pl.loop(0, B)
            def _(e):
                v = fx[e, pl.ds(FW, L)] + erd[e, :]
                v = jnp.where(v >= 0.0, v, 0.2 * v)
                a16 = jnp.exp(v)
                pay[e, pl.ds(FW, L)] = a16
                for k in range(NCH):
                    ah = a16[chunk_heads[k]]
                    pay[e, pl.ds(k * L, L)] = fx[e, pl.ds(k * L, L)] * ah

            pltpu.sync_copy(pay, acc.at[didx], add=True)

        plsc.subcore_barrier()

        pltpu.sync_copy(acc.at[pl.ds(r0, RPT), :], part.at[c, pl.ds(r0, RPT), :])

    return body


@functools.cache
def _make_sc_sweep(FW, PW, chunk_heads):
    mesh = plsc.VectorSubcoreMesh(core_axis_name="c", subcore_axis_name="s",
                                  num_cores=NC, num_subcores=NS)
    return pl.kernel(
        _sc_body(FW, PW, chunk_heads),
        out_type=jax.ShapeDtypeStruct((NC, NP, PW), jnp.float32),
        mesh=mesh,
        scratch_types=_sc_scratch(FW, PW),
        compiler_params=pltpu.CompilerParams(use_tc_tiling_on_sc=False),
    )


BR = 400


def _tc_first(CW):
    def tc_body(x_ref, wfx_ref, wr_ref, fx_ref, ert_ref):
        h = x_ref[...]
        fx_ref[...] = jnp.dot(h, wfx_ref[...], preferred_element_type=jnp.float32)
        ert_ref[...] = jnp.dot(h, wr_ref[...], preferred_element_type=jnp.float32)

    return pl.pallas_call(
        tc_body,
        grid=(N // BR,),
        in_specs=[
            pl.BlockSpec((BR, 128), lambda i: (i, 0)),
            pl.BlockSpec((128, CW), lambda i: (0, 0)),
            pl.BlockSpec((128, L), lambda i: (0, 0)),
        ],
        out_specs=[
            pl.BlockSpec((BR, CW), lambda i: (i, 0)),
            pl.BlockSpec((BR, L), lambda i: (i, 0)),
        ],
        out_shape=[
            jax.ShapeDtypeStruct((N, CW), jnp.float32),
            jax.ShapeDtypeStruct((N, L), jnp.float32),
        ],
    )


def _tc_mid(PWin, CW):
    FWin = PWin - 16

    def tc_body(part_ref, sel_ref, wfx_ref, wr_ref, fx_ref, ert_ref):
        p = part_ref[0] + part_ref[1]
        num = p[:, :FWin]
        den = p[:, FWin:]
        denx = jnp.dot(den, sel_ref[...], preferred_element_type=jnp.float32)
        h = jnp.maximum(num / (denx + 1e-9), 0.0)
        fx_ref[...] = jnp.dot(h, wfx_ref[...], preferred_element_type=jnp.float32)
        ert_ref[...] = jnp.dot(h, wr_ref[...], preferred_element_type=jnp.float32)

    return pl.pallas_call(
        tc_body,
        grid=(N // BR,),
        in_specs=[
            pl.BlockSpec((NC, BR, PWin), lambda i: (0, i, 0)),
            pl.BlockSpec((L, FWin), lambda i: (0, 0)),
            pl.BlockSpec((FWin, CW), lambda i: (0, 0)),
            pl.BlockSpec((FWin, L), lambda i: (0, 0)),
        ],
        out_specs=[
            pl.BlockSpec((BR, CW), lambda i: (i, 0)),
            pl.BlockSpec((BR, L), lambda i: (i, 0)),
        ],
        out_shape=[
            jax.ShapeDtypeStruct((N, CW), jnp.float32),
            jax.ShapeDtypeStruct((N, L), jnp.float32),
        ],
    )


def _tc_final(PWin):
    def tc_body(part_ref, sel_ref, out_ref):
        p = part_ref[0] + part_ref[1]
        den = p[:, PWin - L:]
        denx = jnp.dot(den, sel_ref[...], preferred_element_type=jnp.float32)
        out_ref[...] = p / (denx + 1e-9)

    return pl.pallas_call(
        tc_body,
        grid=(N // BR,),
        in_specs=[
            pl.BlockSpec((NC, BR, PWin), lambda i: (0, i, 0)),
            pl.BlockSpec((L, PWin), lambda i: (0, 0)),
        ],
        out_specs=pl.BlockSpec((BR, PWin), lambda i: (i, 0)),
        out_shape=jax.ShapeDtypeStruct((N, PWin), jnp.float32),
    )


def _head_mat(a):
    H, F = a.shape
    one_hot = jnp.eye(L, dtype=a.dtype)[:H][:, None, :]
    return (a[:, :, None] * one_hot).reshape(H * F, L)


def kernel(inputs, edge_index, W0, al0, ar0, W1, al1, ar1, W2, al2, ar2):
    src = edge_index[0].astype(jnp.int32)
    dst = edge_index[1].astype(jnp.int32)

    Wfx0 = jnp.concatenate([W0, W0 @ _head_mat(al0)], axis=1)
    Wr0 = W0 @ _head_mat(ar0)
    Wfx1 = jnp.concatenate([W1, W1 @ _head_mat(al1)], axis=1)
    Wr1 = W1 @ _head_mat(ar1)
    W2p = jnp.pad(W2, ((0, 0), (0, 8)))
    Wl2 = jnp.pad(W2 @ al2.T, ((0, 0), (0, 15)))
    Wfx2 = jnp.concatenate([W2p, Wl2], axis=1)
    Wr2 = jnp.pad(W2 @ ar2.T, ((0, 0), (0, 15)))
    SEL = _head_mat(jnp.ones((8, 16), jnp.float32)).T
    SEL2 = jnp.zeros((L, 64), jnp.float32).at[0, :48].set(1.0)

    heads8 = tuple(range(8))
    sc01 = _make_sc_sweep(128, 144, heads8)
    sc2 = _make_sc_sweep(48, 64, (0, 0, 0))

    fx0, ert0 = _tc_first(144)(inputs, Wfx0, Wr0)
    part0 = sc01(fx0, ert0, src, dst)
    fx1, ert1 = _tc_mid(144, 144)(part0, SEL, Wfx1, Wr1)
    part1 = sc01(fx1, ert1, src, dst)
    fx2, ert2 = _tc_mid(144, 64)(part1, SEL, Wfx2, Wr2)
    part2 = sc2(fx2, ert2, src, dst)
    out64 = _tc_final(64)(part2, SEL2)
    return out64[:, :40]


# speedup vs baseline: 117.7013x; 2.1185x over previous
import functools

import jax
import jax.numpy as jnp
from jax import lax
from jax.experimental import pallas as pl
from jax.experimental.pallas import tpu as pltpu
from jax.experimental.pallas import tpu_sc as plsc

N = 10000
NP = 10240
E = 320000
NC = 2
NS = 16
L = 16
NW = NC * NS
EPW = E // NW
B = 80
NB = EPW // B
HB = B // 2
RPT = NP // NS


def _sc_scratch(FW, PW):
    CW = FW + 16
    return [
        pltpu.VMEM((B,), jnp.int32),
        pltpu.VMEM((B,), jnp.int32),
        pltpu.VMEM((B,), jnp.int32),
        pltpu.VMEM((B,), jnp.int32),
        pltpu.VMEM((B, CW), jnp.float32),
        pltpu.VMEM((B, L), jnp.float32),
        pltpu.VMEM((B, CW), jnp.float32),
        pltpu.VMEM((B, L), jnp.float32),
        pltpu.VMEM((B, PW), jnp.float32),
        pltpu.VMEM_SHARED((NP, PW), jnp.float32),
        pltpu.SemaphoreType.DMA,
        pltpu.SemaphoreType.DMA,
        pltpu.SemaphoreType.DMA,
        pltpu.SemaphoreType.DMA,
    ]


def _sc_body(FW, PW, chunk_heads):
    NCH = len(chunk_heads)

    def body(featx, ert, srcl, dstl, part,
             sidx0, didx0, sidx1, didx1, fx0, erd0, fx1, erd1, pay, acc,
             sg0, sg1, si0, si1):
        c = lax.axis_index("c")
        s = lax.axis_index("s")
        wid = c * NS + s
        zero16 = jnp.zeros((L,), jnp.float32)
        fxs, erds = (fx0, fx1), (erd0, erd1)
        sidxs, didxs = (sidx0, sidx1), (didx0, didx1)
        gsems, isems = (sg0, sg1), (si0, si1)

        def idx_base(j):
            return wid * EPW + j * B

        def start_idx(j, b):
            pltpu.async_copy(srcl.at[pl.ds(idx_base(j), B)], sidxs[b], isems[b])
            pltpu.async_copy(dstl.at[pl.ds(idx_base(j), B)], didxs[b], isems[b])

        def wait_idx(j, b):
            pltpu.make_async_copy(srcl.at[pl.ds(idx_base(j), B)], sidxs[b], isems[b]).wait()
            pltpu.make_async_copy(dstl.at[pl.ds(idx_base(j), B)], didxs[b], isems[b]).wait()

        def start_gather(b):
            pltpu.async_copy(featx.at[sidxs[b].at[pl.ds(0, HB)]],
                             fxs[b].at[pl.ds(0, HB), :], gsems[b])
            pltpu.async_copy(featx.at[sidxs[b].at[pl.ds(HB, HB)]],
                             fxs[b].at[pl.ds(HB, HB), :], gsems[b])
            pltpu.async_copy(ert.at[didxs[b]], erds[b], gsems[b])

        def wait_gather(b):
            pltpu.make_async_copy(featx.at[sidxs[b].at[pl.ds(0, HB)]],
                                  fxs[b].at[pl.ds(0, HB), :], gsems[b]).wait()
            pltpu.make_async_copy(featx.at[sidxs[b].at[pl.ds(HB, HB)]],
                                  fxs[b].at[pl.ds(HB, HB), :], gsems[b]).wait()
            pltpu.make_async_copy(ert.at[didxs[b]], erds[b], gsems[b]).wait()

        def compute(b):
            fx, erd = fxs[b], erds[b]

            @plsc.parallel_loop(0, B, unroll=4)
            def _(e):
                v = fx[e, pl.ds(FW, L)] + erd[e, :]
                v = jnp.where(v >= 0.0, v, 0.2 * v)
                a16 = jnp.exp(v)
                pay[e, pl.ds(FW, L)] = a16
                for k in range(NCH):
                    ah = a16[chunk_heads[k]]
                    pay[e, pl.ds(k * L, L)] = fx[e, pl.ds(k * L, L)] * ah

            pltpu.sync_copy(pay, acc.at[didxs[b]], add=True)

        start_idx(0, 0)
        wait_idx(0, 0)
        start_gather(0)
        start_idx(1, 1)

        @pl.loop(0, B)
        def _(r):
            for k in range(PW // L):
                pay[r, pl.ds(k * L, L)] = zero16

        r0 = s * RPT

        @pl.loop(0, RPT // B)
        def _(i):
            pltpu.sync_copy(pay, acc.at[pl.ds(r0 + i * B, B), :])

        plsc.subcore_barrier()

        @pl.loop(0, NB - 1, step=2)
        def _(jj):
            wait_idx(jj + 1, 1)
            start_gather(1)
            wait_gather(0)
            compute(0)

            @pl.when(jj + 2 < NB)
            def _():
                start_idx(jj + 2, 0)

            wait_gather(1)
            compute(1)

            @pl.when(jj + 2 < NB)
            def _():
                wait_idx(jj + 2, 0)
                start_gather(0)

            @pl.when(jj + 3 < NB)
            def _():
                start_idx(jj + 3, 1)

        wait_gather(0)
        compute(0)

        plsc.subcore_barrier()

        pltpu.sync_copy(acc.at[pl.ds(r0, RPT), :], part.at[c, pl.ds(r0, RPT), :])

    return body


@functools.cache
def _make_sc_sweep(FW, PW, chunk_heads):
    mesh = plsc.VectorSubcoreMesh(core_axis_name="c", subcore_axis_name="s",
                                  num_cores=NC, num_subcores=NS)
    return pl.kernel(
        _sc_body(FW, PW, chunk_heads),
        out_type=jax.ShapeDtypeStruct((NC, NP, PW), jnp.float32),
        mesh=mesh,
        scratch_types=_sc_scratch(FW, PW),
        compiler_params=pltpu.CompilerParams(use_tc_tiling_on_sc=False),
    )


BR = 400


def _tc_first(CW):
    def tc_body(x_ref, wfx_ref, wr_ref, fx_ref, ert_ref):
        h = x_ref[...]
        fx_ref[...] = jnp.dot(h, wfx_ref[...], preferred_element_type=jnp.float32)
        ert_ref[...] = jnp.dot(h, wr_ref[...], preferred_element_type=jnp.float32)

    return pl.pallas_call(
        tc_body,
        grid=(N // BR,),
        in_specs=[
            pl.BlockSpec((BR, 128), lambda i: (i, 0)),
            pl.BlockSpec((128, CW), lambda i: (0, 0)),
            pl.BlockSpec((128, L), lambda i: (0, 0)),
        ],
        out_specs=[
            pl.BlockSpec((BR, CW), lambda i: (i, 0)),
            pl.BlockSpec((BR, L), lambda i: (i, 0)),
        ],
        out_shape=[
            jax.ShapeDtypeStruct((N, CW), jnp.float32),
            jax.ShapeDtypeStruct((N, L), jnp.float32),
        ],
    )


def _tc_mid(PWin, CW):
    FWin = PWin - 16

    def tc_body(part_ref, sel_ref, wfx_ref, wr_ref, fx_ref, ert_ref):
        p = part_ref[0] + part_ref[1]
        num = p[:, :FWin]
        den = p[:, FWin:]
        denx = jnp.dot(den, sel_ref[...], preferred_element_type=jnp.float32)
        h = jnp.maximum(num / (denx + 1e-9), 0.0)
        fx_ref[...] = jnp.dot(h, wfx_ref[...], preferred_element_type=jnp.float32)
        ert_ref[...] = jnp.dot(h, wr_ref[...], preferred_element_type=jnp.float32)

    return pl.pallas_call(
        tc_body,
        grid=(N // BR,),
        in_specs=[
            pl.BlockSpec((NC, BR, PWin), lambda i: (0, i, 0)),
            pl.BlockSpec((L, FWin), lambda i: (0, 0)),
            pl.BlockSpec((FWin, CW), lambda i: (0, 0)),
            pl.BlockSpec((FWin, L), lambda i: (0, 0)),
        ],
        out_specs=[
            pl.BlockSpec((BR, CW), lambda i: (i, 0)),
            pl.BlockSpec((BR, L), lambda i: (i, 0)),
        ],
        out_shape=[
            jax.ShapeDtypeStruct((N, CW), jnp.float32),
            jax.ShapeDtypeStruct((N, L), jnp.float32),
        ],
    )


def _tc_final(PWin):
    def tc_body(part_ref, sel_ref, out_ref):
        p = part_ref[0] + part_ref[1]
        den = p[:, PWin - L:]
        denx = jnp.dot(den, sel_ref[...], preferred_element_type=jnp.float32)
        out_ref[...] = p / (denx + 1e-9)

    return pl.pallas_call(
        tc_body,
        grid=(N // BR,),
        in_specs=[
            pl.BlockSpec((NC, BR, PWin), lambda i: (0, i, 0)),
            pl.BlockSpec((L, PWin), lambda i: (0, 0)),
        ],
        out_specs=pl.BlockSpec((BR, PWin), lambda i: (i, 0)),
        out_shape=jax.ShapeDtypeStruct((N, PWin), jnp.float32),
    )


def _head_mat(a):
    H, F = a.shape
    one_hot = jnp.eye(L, dtype=a.dtype)[:H][:, None, :]
    return (a[:, :, None] * one_hot).reshape(H * F, L)


def kernel(inputs, edge_index, W0, al0, ar0, W1, al1, ar1, W2, al2, ar2):
    src = edge_index[0].astype(jnp.int32)
    dst = edge_index[1].astype(jnp.int32)

    Wfx0 = jnp.concatenate([W0, W0 @ _head_mat(al0)], axis=1)
    Wr0 = W0 @ _head_mat(ar0)
    Wfx1 = jnp.concatenate([W1, W1 @ _head_mat(al1)], axis=1)
    Wr1 = W1 @ _head_mat(ar1)
    W2p = jnp.pad(W2, ((0, 0), (0, 8)))
    Wl2 = jnp.pad(W2 @ al2.T, ((0, 0), (0, 15)))
    Wfx2 = jnp.concatenate([W2p, Wl2], axis=1)
    Wr2 = jnp.pad(W2 @ ar2.T, ((0, 0), (0, 15)))
    SEL = _head_mat(jnp.ones((8, 16), jnp.float32)).T
    SEL2 = jnp.zeros((L, 64), jnp.float32).at[0, :48].set(1.0)

    heads8 = tuple(range(8))
    sc01 = _make_sc_sweep(128, 144, heads8)
    sc2 = _make_sc_sweep(48, 64, (0, 0, 0))

    fx0, ert0 = _tc_first(144)(inputs, Wfx0, Wr0)
    part0 = sc01(fx0, ert0, src, dst)
    fx1, ert1 = _tc_mid(144, 144)(part0, SEL, Wfx1, Wr1)
    part1 = sc01(fx1, ert1, src, dst)
    fx2, ert2 = _tc_mid(144, 64)(part1, SEL, Wfx2, Wr2)
    part2 = sc2(fx2, ert2, src, dst)
    out64 = _tc_final(64)(part2, SEL2)
    return out64[:, :40]


# speedup vs baseline: 123.7681x; 1.0515x over previous
import functools

import jax
import jax.numpy as jnp
from jax import lax
from jax.experimental import pallas as pl
from jax.experimental.pallas import tpu as pltpu
from jax.experimental.pallas import tpu_sc as plsc

N = 10000
NP = 10240
E = 320000
NC = 2
NS = 16
L = 16
NW = NC * NS
EPW = E // NW
B = 80
NB = EPW // B
HB = B // 2
RPT = NP // NS


def _sc_scratch(FW, PW):
    CW = FW + 16
    return [
        pltpu.VMEM((B,), jnp.int32),
        pltpu.VMEM((2, HB), jnp.int32),
        pltpu.VMEM((B,), jnp.int32),
        pltpu.VMEM((2, HB), jnp.int32),
        pltpu.VMEM((B, CW), jnp.float32),
        pltpu.VMEM((B, L), jnp.float32),
        pltpu.VMEM((B, CW), jnp.float32),
        pltpu.VMEM((B, L), jnp.float32),
        pltpu.VMEM((B, PW), jnp.float32),
        pltpu.VMEM((2, HB), jnp.int32),
        pltpu.VMEM((2, HB), jnp.int32),
        pltpu.VMEM_SHARED((NP, PW), jnp.float32),
        pltpu.SemaphoreType.DMA,
        pltpu.SemaphoreType.DMA,
        pltpu.SemaphoreType.DMA,
        pltpu.SemaphoreType.DMA,
        pltpu.SemaphoreType.DMA,
        pltpu.SemaphoreType.DMA,
        pltpu.SemaphoreType.DMA,
        pltpu.SemaphoreType.DMA,
    ]


def _sc_body(FW, PW, chunk_heads):
    NCH = len(chunk_heads)

    def body(featx, ert, srcl, dstl, part,
             sidx0, didx0, sidx1, didx1, fx0, erd0, fx1, erd1, pay,
             scidx0, scidx1, acc,
             sg0, sg1, si0, si1, ss0, ss1, sc0, sc1):
        c = lax.axis_index("c")
        s = lax.axis_index("s")
        wid = c * NS + s
        zero16 = jnp.zeros((L,), jnp.float32)
        fxs, erds = (fx0, fx1), (erd0, erd1)
        sidxs, didxs = (sidx0, sidx1), (didx0, didx1)
        gsems, isems, ssems = (sg0, sg1), (si0, si1), (ss0, ss1)
        scidxs, scsems = (scidx0, scidx1), (sc0, sc1)

        def idx_base(j):
            return wid * EPW + j * B

        def start_idx(j, b):
            pltpu.async_copy(srcl.at[pl.ds(idx_base(j), B)], sidxs[b], isems[b])
            pltpu.async_copy(dstl.at[pl.ds(idx_base(j), HB)], didxs[b].at[0], isems[b])
            pltpu.async_copy(dstl.at[pl.ds(idx_base(j) + HB, HB)], didxs[b].at[1], isems[b])

        def wait_idx(j, b):
            pltpu.make_async_copy(srcl.at[pl.ds(idx_base(j), B)], sidxs[b], isems[b]).wait()
            pltpu.make_async_copy(dstl.at[pl.ds(idx_base(j), HB)], didxs[b].at[0], isems[b]).wait()
            pltpu.make_async_copy(dstl.at[pl.ds(idx_base(j) + HB, HB)], didxs[b].at[1], isems[b]).wait()

        def start_gather(b):
            pltpu.async_copy(featx.at[sidxs[b].at[pl.ds(0, HB)]],
                             fxs[b].at[pl.ds(0, HB), :], gsems[b])
            pltpu.async_copy(featx.at[sidxs[b].at[pl.ds(HB, HB)]],
                             fxs[b].at[pl.ds(HB, HB), :], gsems[b])
            pltpu.async_copy(ert.at[didxs[b].at[0]], erds[b].at[pl.ds(0, HB), :], gsems[b])
            pltpu.async_copy(ert.at[didxs[b].at[1]], erds[b].at[pl.ds(HB, HB), :], gsems[b])

        def wait_gather(b):
            pltpu.make_async_copy(featx.at[sidxs[b].at[pl.ds(0, HB)]],
                                  fxs[b].at[pl.ds(0, HB), :], gsems[b]).wait()
            pltpu.make_async_copy(featx.at[sidxs[b].at[pl.ds(HB, HB)]],
                                  fxs[b].at[pl.ds(HB, HB), :], gsems[b]).wait()
            pltpu.make_async_copy(ert.at[didxs[b].at[0]], erds[b].at[pl.ds(0, HB), :], gsems[b]).wait()
            pltpu.make_async_copy(ert.at[didxs[b].at[1]], erds[b].at[pl.ds(HB, HB), :], gsems[b]).wait()

        def wait_scatter(b, h):
            pltpu.make_async_copy(pay.at[pl.ds(h * HB, HB), :],
                                  acc.at[scidxs[b].at[h]], ssems[h]).wait()

        def half_loop(fx, erd, h):
            @plsc.parallel_loop(h * HB, (h + 1) * HB, unroll=4)
            def _(e):
                v = fx[e, pl.ds(FW, L)] + erd[e, :]
                v = jnp.where(v >= 0.0, v, 0.2 * v)
                a16 = jnp.exp(v)
                pay[e, pl.ds(FW, L)] = a16
                for k in range(NCH):
                    ah = a16[chunk_heads[k]]
                    pay[e, pl.ds(k * L, L)] = fx[e, pl.ds(k * L, L)] * ah

        def compute(j, b):
            fx, erd = fxs[b], erds[b]
            pltpu.async_copy(dstl.at[pl.ds(idx_base(j), HB)], scidxs[b].at[0], scsems[b])
            pltpu.async_copy(dstl.at[pl.ds(idx_base(j) + HB, HB)], scidxs[b].at[1], scsems[b])

            @pl.when(j > 0)
            def _():
                wait_scatter(1 - b, 0)

            half_loop(fx, erd, 0)
            pltpu.make_async_copy(dstl.at[pl.ds(idx_base(j), HB)], scidxs[b].at[0], scsems[b]).wait()
            pltpu.make_async_copy(dstl.at[pl.ds(idx_base(j) + HB, HB)], scidxs[b].at[1], scsems[b]).wait()
            pltpu.async_copy(pay.at[pl.ds(0, HB), :],
                             acc.at[scidxs[b].at[0]], ssems[0], add=True)

            @pl.when(j > 0)
            def _():
                wait_scatter(1 - b, 1)

            half_loop(fx, erd, 1)
            pltpu.async_copy(pay.at[pl.ds(HB, HB), :],
                             acc.at[scidxs[b].at[1]], ssems[1], add=True)

        start_idx(0, 0)
        wait_idx(0, 0)
        start_gather(0)
        start_idx(1, 1)

        @pl.loop(0, B)
        def _(r):
            for k in range(PW // L):
                pay[r, pl.ds(k * L, L)] = zero16

        r0 = s * RPT

        @pl.loop(0, RPT // B)
        def _(i):
            pltpu.sync_copy(pay, acc.at[pl.ds(r0 + i * B, B), :])

        plsc.subcore_barrier()

        @pl.loop(0, NB - 1, step=2)
        def _(jj):
            wait_idx(jj + 1, 1)
            start_gather(1)
            wait_gather(0)
            compute(jj, 0)

            @pl.when(jj + 2 < NB)
            def _():
                start_idx(jj + 2, 0)

            wait_gather(1)
            compute(jj + 1, 1)

            @pl.when(jj + 2 < NB)
            def _():
                wait_idx(jj + 2, 0)
                start_gather(0)

            @pl.when(jj + 3 < NB)
            def _():
                start_idx(jj + 3, 1)

        wait_gather(0)
        compute(NB - 1, 0)
        wait_scatter(0, 0)
        wait_scatter(0, 1)

        plsc.subcore_barrier()

        pltpu.sync_copy(acc.at[pl.ds(r0, RPT), :], part.at[c, pl.ds(r0, RPT), :])

    return body


@functools.cache
def _make_sc_sweep(FW, PW, chunk_heads):
    mesh = plsc.VectorSubcoreMesh(core_axis_name="c", subcore_axis_name="s",
                                  num_cores=NC, num_subcores=NS)
    return pl.kernel(
        _sc_body(FW, PW, chunk_heads),
        out_type=jax.ShapeDtypeStruct((NC, NP, PW), jnp.float32),
        mesh=mesh,
        scratch_types=_sc_scratch(FW, PW),
        compiler_params=pltpu.CompilerParams(use_tc_tiling_on_sc=False),
    )


BR = 400


def _tc_first(CW):
    def tc_body(x_ref, wfx_ref, wr_ref, fx_ref, ert_ref):
        h = x_ref[...]
        fx_ref[...] = jnp.dot(h, wfx_ref[...], preferred_element_type=jnp.float32)
        ert_ref[...] = jnp.dot(h, wr_ref[...], preferred_element_type=jnp.float32)

    return pl.pallas_call(
        tc_body,
        grid=(N // BR,),
        in_specs=[
            pl.BlockSpec((BR, 128), lambda i: (i, 0)),
            pl.BlockSpec((128, CW), lambda i: (0, 0)),
            pl.BlockSpec((128, L), lambda i: (0, 0)),
        ],
        out_specs=[
            pl.BlockSpec((BR, CW), lambda i: (i, 0)),
            pl.BlockSpec((BR, L), lambda i: (i, 0)),
        ],
        out_shape=[
            jax.ShapeDtypeStruct((N, CW), jnp.float32),
            jax.ShapeDtypeStruct((N, L), jnp.float32),
        ],
    )


def _tc_mid(PWin, CW):
    FWin = PWin - 16

    def tc_body(part_ref, sel_ref, wfx_ref, wr_ref, fx_ref, ert_ref):
        p = part_ref[0] + part_ref[1]
        num = p[:, :FWin]
        den = p[:, FWin:]
        denx = jnp.dot(den, sel_ref[...], preferred_element_type=jnp.float32)
        h = jnp.maximum(num / (denx + 1e-9), 0.0)
        fx_ref[...] = jnp.dot(h, wfx_ref[...], preferred_element_type=jnp.float32)
        ert_ref[...] = jnp.dot(h, wr_ref[...], preferred_element_type=jnp.float32)

    return pl.pallas_call(
        tc_body,
        grid=(N // BR,),
        in_specs=[
            pl.BlockSpec((NC, BR, PWin), lambda i: (0, i, 0)),
            pl.BlockSpec((L, FWin), lambda i: (0, 0)),
            pl.BlockSpec((FWin, CW), lambda i: (0, 0)),
            pl.BlockSpec((FWin, L), lambda i: (0, 0)),
        ],
        out_specs=[
            pl.BlockSpec((BR, CW), lambda i: (i, 0)),
            pl.BlockSpec((BR, L), lambda i: (i, 0)),
        ],
        out_shape=[
            jax.ShapeDtypeStruct((N, CW), jnp.float32),
            jax.ShapeDtypeStruct((N, L), jnp.float32),
        ],
    )


def _tc_final(PWin):
    def tc_body(part_ref, sel_ref, out_ref):
        p = part_ref[0] + part_ref[1]
        den = p[:, PWin - L:]
        denx = jnp.dot(den, sel_ref[...], preferred_element_type=jnp.float32)
        out_ref[...] = p / (denx + 1e-9)

    return pl.pallas_call(
        tc_body,
        grid=(N // BR,),
        in_specs=[
            pl.BlockSpec((NC, BR, PWin), lambda i: (0, i, 0)),
            pl.BlockSpec((L, PWin), lambda i: (0, 0)),
        ],
        out_specs=pl.BlockSpec((BR, PWin), lambda i: (i, 0)),
        out_shape=jax.ShapeDtypeStruct((N, PWin), jnp.float32),
    )


def _head_mat(a):
    H, F = a.shape
    one_hot = jnp.eye(L, dtype=a.dtype)[:H][:, None, :]
    return (a[:, :, None] * one_hot).reshape(H * F, L)


def kernel(inputs, edge_index, W0, al0, ar0, W1, al1, ar1, W2, al2, ar2):
    src = edge_index[0].astype(jnp.int32)
    dst = edge_index[1].astype(jnp.int32)

    Wfx0 = jnp.concatenate([W0, W0 @ _head_mat(al0)], axis=1)
    Wr0 = W0 @ _head_mat(ar0)
    Wfx1 = jnp.concatenate([W1, W1 @ _head_mat(al1)], axis=1)
    Wr1 = W1 @ _head_mat(ar1)
    W2p = jnp.pad(W2, ((0, 0), (0, 8)))
    Wl2 = jnp.pad(W2 @ al2.T, ((0, 0), (0, 15)))
    Wfx2 = jnp.concatenate([W2p, Wl2], axis=1)
    Wr2 = jnp.pad(W2 @ ar2.T, ((0, 0), (0, 15)))
    SEL = _head_mat(jnp.ones((8, 16), jnp.float32)).T
    SEL2 = jnp.zeros((L, 64), jnp.float32).at[0, :48].set(1.0)

    heads8 = tuple(range(8))
    sc01 = _make_sc_sweep(128, 144, heads8)
    sc2 = _make_sc_sweep(48, 64, (0, 0, 0))

    fx0, ert0 = _tc_first(144)(inputs, Wfx0, Wr0)
    part0 = sc01(fx0, ert0, src, dst)
    fx1, ert1 = _tc_mid(144, 144)(part0, SEL, Wfx1, Wr1)
    part1 = sc01(fx1, ert1, src, dst)
    fx2, ert2 = _tc_mid(144, 64)(part1, SEL, Wfx2, Wr2)
    part2 = sc2(fx2, ert2, src, dst)
    out64 = _tc_final(64)(part2, SEL2)
    return out64[:, :40]
